# Initial kernel scaffold; baseline (speedup 1.0000x reference)
import jax
import jax.numpy as jnp
from jax.experimental import pallas as pl


def kernel(confidence, errors):
    raise NotImplementedError("write your pallas kernel here")


# speedup vs baseline: 39.9692x; 39.9692x over previous
import functools

import jax
import jax.numpy as jnp
import numpy as np
from jax import lax
from jax.experimental import pallas as pl
from jax.experimental.pallas import tpu as pltpu
from jax.experimental.pallas import tpu_sc as plsc

N_BINS = 15
B = 1024
NC, NS, L = 2, 16, 16
NW = NC * NS
TBL = L * B
NCH = 5

_mesh = plsc.VectorSubcoreMesh(core_axis_name="c", subcore_axis_name="s")


def _make_sc_hist(n):
    per_tile = n // NW
    chunk = per_tile // NCH
    full = chunk // L
    tail = chunk - full * L
    assert per_tile * NW == n and chunk * NCH == per_tile and chunk % 8 == 0

    @functools.partial(
        pl.kernel,
        out_type=(
            jax.ShapeDtypeStruct((NW, TBL), jnp.float32),
            jax.ShapeDtypeStruct((NW, TBL), jnp.float32),
            jax.ShapeDtypeStruct((NW, TBL), jnp.float32),
        ),
        mesh=_mesh,
        compiler_params=pltpu.CompilerParams(needs_layout_passes=False),
        scratch_types=[
            pltpu.VMEM((chunk + L,), jnp.float32),
            pltpu.VMEM((chunk + L,), jnp.float32),
            pltpu.VMEM((TBL,), jnp.float32),
            pltpu.VMEM((TBL,), jnp.float32),
            pltpu.VMEM((TBL,), jnp.float32),
        ],
    )
    def sc_hist(conf_hbm, err_hbm, ocnt, oconf, oerr, conf_v, err_v,
                hcnt, hconf, herr):
        wid = lax.axis_index("s") * NC + lax.axis_index("c")
        base = wid * per_tile

        zeros = jnp.zeros((L,), jnp.float32)

        def zbody(i, _):
            hcnt[pl.ds(i * L, L)] = zeros
            hconf[pl.ds(i * L, L)] = zeros
            herr[pl.ds(i * L, L)] = zeros
            return 0

        lax.fori_loop(0, TBL // L, zbody, 0)

        lane_base = lax.iota(jnp.int32, L) * B
        ones = jnp.ones((L,), jnp.float32)
        full_mask = lax.iota(jnp.int32, L) < L
        tail_mask = lax.iota(jnp.int32, L) < tail

        def accum(c, e, mask):
            k = (c * float(B)).astype(jnp.int32)
            k = jnp.minimum(jnp.maximum(k, 0), B - 1)
            addr = k + lane_base
            plsc.addupdate_scatter(hcnt, [addr], ones, mask=mask)
            plsc.addupdate_scatter(hconf, [addr], c, mask=mask)
            plsc.addupdate_scatter(herr, [addr], e, mask=mask)

        def chunk_body(ci, _):
            start = base + ci * chunk
            pltpu.sync_copy(conf_hbm.at[pl.ds(start, chunk)],
                            conf_v.at[pl.ds(0, chunk)])
            pltpu.sync_copy(err_hbm.at[pl.ds(start, chunk)],
                            err_v.at[pl.ds(0, chunk)])

            def vbody(i, _):
                accum(conf_v[pl.ds(i * L, L)], err_v[pl.ds(i * L, L)],
                      full_mask)
                return 0

            lax.fori_loop(0, full, vbody, 0)
            if tail:
                accum(conf_v[pl.ds(full * L, L)], err_v[pl.ds(full * L, L)],
                      mask=tail_mask)
            return 0

        lax.fori_loop(0, NCH, chunk_body, 0)

        pltpu.sync_copy(hcnt, ocnt.at[wid])
        pltpu.sync_copy(hconf, oconf.at[wid])
        pltpu.sync_copy(herr, oerr.at[wid])

    return sc_hist


def _make_tc_finalize():
    def body(cnt_ref, conf_ref, err_ref, rk_ref, wdt_ref, out_ref):
        cnt = jnp.sum(cnt_ref[...], axis=0, keepdims=True)
        sconf = jnp.sum(conf_ref[...], axis=0, keepdims=True)
        serr = jnp.sum(err_ref[...], axis=0, keepdims=True)

        ii = lax.broadcasted_iota(jnp.int32, (B, B), 0)
        jj = lax.broadcasted_iota(jnp.int32, (B, B), 1)
        tri = (ii < jj).astype(jnp.float32)
        cc_excl = jnp.dot(cnt, tri, preferred_element_type=jnp.float32)

        safe = jnp.maximum(cnt, 1.0)
        mc = sconf / safe
        me = serr / safe

        rk = rk_ref[...]
        w = jnp.clip(rk - cc_excl, 0.0, cnt)
        fc = jnp.sum(w * mc, axis=1, keepdims=True)
        fe = jnp.sum(w * me, axis=1, keepdims=True)

        sk = fc[1:, :] - fc[:-1, :]
        ek = fe[1:, :] - fe[:-1, :]
        wdt = wdt_ref[...]
        val = jnp.abs(sk / wdt - (1.0 - ek / wdt))
        out_ref[...] = (jnp.sum(val) / float(N_BINS)).reshape(1, 1)

    return pl.pallas_call(
        body,
        out_shape=jax.ShapeDtypeStruct((1, 1), jnp.float32),
    )


def kernel(confidence, errors):
    n = confidence.shape[0]
    ranks = np.linspace(0.0, float(n), N_BINS + 1).astype(np.int64)
    rk = jnp.asarray(ranks.astype(np.float32)).reshape(N_BINS + 1, 1)
    wdt = jnp.asarray(np.diff(ranks).astype(np.float32)).reshape(N_BINS, 1)
    ocnt, oconf, oerr = _make_sc_hist(n)(confidence, errors)
    cnt2 = ocnt.reshape(NW * L, B)
    conf2 = oconf.reshape(NW * L, B)
    err2 = oerr.reshape(NW * L, B)
    out = _make_tc_finalize()(cnt2, conf2, err2, rk, wdt)
    return out.reshape(())


# speedup vs baseline: 122.9505x; 3.0761x over previous
import functools

import jax
import jax.numpy as jnp
import numpy as np
from jax import lax
from jax.experimental import pallas as pl
from jax.experimental.pallas import tpu as pltpu
from jax.experimental.pallas import tpu_sc as plsc

N_BINS = 15
B = 512
NC, NS, L = 2, 16, 16
NW = NC * NS
TBL = L * B
NCH = 5

_mesh = plsc.VectorSubcoreMesh(core_axis_name="c", subcore_axis_name="s")


def _make_sc_hist(n):
    per_tile = n // NW
    chunk = 20832
    nch = per_tile // chunk
    full = chunk // L
    tail = per_tile - nch * chunk
    assert per_tile * NW == n and chunk % 8 == 0 and chunk % L == 0
    assert 0 < tail < L and (per_tile - L) % 8 == 0

    @functools.partial(
        pl.kernel,
        out_type=(
            jax.ShapeDtypeStruct((NW * L, B), jnp.float32),
            jax.ShapeDtypeStruct((NW * L, B), jnp.float32),
        ),
        mesh=_mesh,
        compiler_params=pltpu.CompilerParams(needs_layout_passes=False),
        scratch_types=[
            pltpu.VMEM((chunk,), jnp.float32),
            pltpu.VMEM((chunk,), jnp.float32),
            pltpu.VMEM((chunk,), jnp.float32),
            pltpu.VMEM((chunk,), jnp.float32),
            pltpu.VMEM((L, B), jnp.float32),
            pltpu.VMEM((L, B), jnp.float32),
            pltpu.VMEM((L,), jnp.float32),
            pltpu.VMEM((L,), jnp.float32),
            pltpu.SemaphoreType.DMA,
            pltpu.SemaphoreType.DMA,
            pltpu.SemaphoreType.DMA,
            pltpu.SemaphoreType.DMA,
        ],
    )
    def sc_hist(conf_hbm, err_hbm, os1, oerr, conf_v0, conf_v1,
                err_v0, err_v1, hs1, herr, ctail, etail_v,
                sc0, sc1, se0, se1):
        wid = lax.axis_index("s") * NC + lax.axis_index("c")
        base = wid * per_tile
        bufc = (conf_v0, conf_v1)
        bufe = (err_v0, err_v1)
        semc = (sc0, sc1)
        seme = (se0, se1)

        def copies(ci, b):
            start = base + ci * chunk
            cc = pltpu.make_async_copy(
                conf_hbm.at[pl.ds(start, chunk)], bufc[b], semc[b])
            ee = pltpu.make_async_copy(
                err_hbm.at[pl.ds(start, chunk)], bufe[b], seme[b])
            return cc, ee

        cc, ee = copies(0, 0)
        cc.start()
        ee.start()

        zeros = jnp.zeros((L,), jnp.float32)

        def zbody(i, _):
            for r in range(L):
                hs1[r, pl.ds(i * L, L)] = zeros
                herr[r, pl.ds(i * L, L)] = zeros
            return 0

        lax.fori_loop(0, B // L, zbody, 0)

        lanes = lax.iota(jnp.int32, L)
        full_mask = lanes < L
        tail_mask = lanes >= (L - tail)
        c1 = 1.0 / 16384.0
        c2 = 1.0 - 0.5 / 16384.0

        def accum(c, e, mask, clamp=False):
            cb = c * float(B)
            k = cb.astype(jnp.int32)
            if clamp:
                k = jnp.minimum(jnp.maximum(k, 0), B - 1)
            kf = k.astype(jnp.float32)
            v1 = (cb - kf) * c1 + c2
            plsc.addupdate_scatter(hs1, [lanes, k], v1, mask=mask)
            plsc.addupdate_scatter(herr, [lanes, k], e, mask=mask)

        def process(cv, ev, nvreg):
            @plsc.parallel_loop(0, nvreg, unroll=6)
            def _(i):
                accum(cv[pl.ds(i * L, L)], ev[pl.ds(i * L, L)], full_mask)

        for ci in range(nch):
            b = ci % 2
            if ci + 1 < nch:
                cn, en = copies(ci + 1, 1 - b)
                cn.start()
                en.start()
            cw, ew = copies(ci, b)
            cw.wait()
            ew.wait()
            process(bufc[b], bufe[b], full)

        estart = base + per_tile - L
        pltpu.sync_copy(conf_hbm.at[pl.ds(estart, L)], ctail)
        pltpu.sync_copy(err_hbm.at[pl.ds(estart, L)], etail_v)
        accum(ctail[...], etail_v[...], tail_mask, clamp=True)

        pltpu.sync_copy(hs1, os1.at[pl.ds(wid * L, L)])
        pltpu.sync_copy(herr, oerr.at[pl.ds(wid * L, L)])

    return sc_hist


def _make_tc_finalize():
    def body(s1_ref, err_ref, rk_ref, wdt_ref, out_ref):
        s1 = jnp.sum(s1_ref[...], axis=0, keepdims=True)
        serr = jnp.sum(err_ref[...], axis=0, keepdims=True)
        cnt = jnp.floor(s1 + 0.5)
        dev = (s1 - cnt) * 16384.0
        jc = lax.broadcasted_iota(jnp.int32, (1, B), 1).astype(jnp.float32) + 0.5
        sconf = (jc * cnt + dev) * (1.0 / B)

        ii = lax.broadcasted_iota(jnp.int32, (B, B), 0)
        jj = lax.broadcasted_iota(jnp.int32, (B, B), 1)
        tri = (ii < jj).astype(jnp.float32)
        cc_excl = jnp.dot(cnt, tri, preferred_element_type=jnp.float32)

        safe = jnp.maximum(cnt, 1.0)
        mc = sconf / safe
        me = serr / safe

        rk = rk_ref[...]
        w = jnp.clip(rk - cc_excl, 0.0, cnt)
        fc = jnp.sum(w * mc, axis=1, keepdims=True)
        fe = jnp.sum(w * me, axis=1, keepdims=True)

        sk = fc[1:, :] - fc[:-1, :]
        ek = fe[1:, :] - fe[:-1, :]
        wdt = wdt_ref[...]
        val = jnp.abs(sk / wdt - (1.0 - ek / wdt))
        out_ref[...] = (jnp.sum(val) / float(N_BINS)).reshape(1, 1)

    return pl.pallas_call(
        body,
        out_shape=jax.ShapeDtypeStruct((1, 1), jnp.float32),
    )


def kernel(confidence, errors):
    n = confidence.shape[0]
    ranks = np.linspace(0.0, float(n), N_BINS + 1).astype(np.int64)
    rk = jnp.asarray(ranks.astype(np.float32)).reshape(N_BINS + 1, 1)
    wdt = jnp.asarray(np.diff(ranks).astype(np.float32)).reshape(N_BINS, 1)
    os1, oerr = _make_sc_hist(n)(confidence, errors)
    out = _make_tc_finalize()(os1, oerr, rk, wdt)
    return out.reshape(())
